# baseline (speedup 1.0000x reference)
import functools

import jax
import jax.numpy as jnp
import numpy as np
from jax import lax
from jax.experimental import pallas as pl
from jax.experimental.pallas import tpu as pltpu
from jax.experimental.pallas import tpu_sc as plsc

EMBED_DIM = 256
NUM_HEADS = 8
HEAD_DIM = EMBED_DIM // NUM_HEADS
NUM_LEVELS = 3
NUM_POINTS = 4
LEVEL_SHAPES = ((64, 64), (32, 32), (16, 16))

NCOL = NUM_POINTS * NUM_HEADS * NUM_LEVELS

NC, NS = 2, 16
NW = NC * NS
NBUF = 4


def _col_constants(batch):
    wf = np.zeros((1, NCOL), np.float32)
    hf = np.zeros((1, NCOL), np.float32)
    wi = np.zeros((1, NCOL), np.int32)
    hi = np.zeros((1, NCOL), np.int32)
    ab = np.zeros((1, NCOL), np.int32)
    hw8 = np.zeros((1, NCOL), np.int32)
    base = 0
    level_base = []
    for (H, W) in LEVEL_SHAPES:
        level_base.append(base)
        base += batch * NUM_HEADS * H * W
    for col in range(NCOL):
        l = col % NUM_LEVELS
        h = (col // NUM_LEVELS) % NUM_HEADS
        H, W = LEVEL_SHAPES[l]
        wf[0, col] = W
        hf[0, col] = H
        wi[0, col] = W
        hi[0, col] = H
        ab[0, col] = level_base[l] + h * H * W
        hw8[0, col] = NUM_HEADS * H * W
    return wf, hf, wi, hi, ab, hw8


def _k1_body(q_ref, r_ref, wo_ref, bo_ref, wa_ref, ba_ref, wf_ref, hf_ref,
             wi_ref, hi_ref, ab_ref, hw8_ref, idx_ref, w_ref):
    n = q_ref.shape[0]
    q = q_ref[...]
    off = jnp.dot(q, wo_ref[...], preferred_element_type=jnp.float32) + bo_ref[...]
    att = jnp.dot(q, wa_ref[...], preferred_element_type=jnp.float32) + ba_ref[...]

    g = NUM_HEADS * NUM_LEVELS
    a0, a1, a2, a3 = att[:, 0:g], att[:, g:2 * g], att[:, 2 * g:3 * g], att[:, 3 * g:4 * g]
    m = jnp.maximum(jnp.maximum(a0, a1), jnp.maximum(a2, a3))
    e0 = jnp.exp(a0 - m)
    e1 = jnp.exp(a1 - m)
    e2 = jnp.exp(a2 - m)
    e3 = jnp.exp(a3 - m)
    s = e0 + e1 + e2 + e3
    attn = jnp.concatenate([e0 / s, e1 / s, e2 / s, e3 / s], axis=1)

    refx = r_ref[:, 0:1]
    refy = r_ref[:, 1:2]
    locx = refx + jnp.tanh(off[:, 0:NCOL]) * 0.5
    locy = refy + jnp.tanh(off[:, NCOL:2 * NCOL]) * 0.5

    wf = wf_ref[...]
    hf = hf_ref[...]
    wi = wi_ref[...]
    hi = hi_ref[...]

    gx = locx * 2.0 - 1.0
    gy = locy * 2.0 - 1.0
    ix = ((gx + 1.0) * wf - 1.0) * 0.5
    iy = ((gy + 1.0) * hf - 1.0) * 0.5
    x0f = jnp.floor(ix)
    y0f = jnp.floor(iy)
    wx1 = ix - x0f
    wx0 = 1.0 - wx1
    wy1 = iy - y0f
    wy0 = 1.0 - wy1
    x0 = x0f.astype(jnp.int32)
    y0 = y0f.astype(jnp.int32)
    x1 = x0 + 1
    y1 = y0 + 1

    zero = jnp.zeros_like(x0)
    vx0 = ((x0 >= 0) & (x0 < wi)).astype(jnp.float32)
    vx1 = ((x1 >= 0) & (x1 < wi)).astype(jnp.float32)
    vy0 = ((y0 >= 0) & (y0 < hi)).astype(jnp.float32)
    vy1 = ((y1 >= 0) & (y1 < hi)).astype(jnp.float32)
    x0c = jnp.minimum(jnp.maximum(x0, zero), wi - 1)
    x1c = jnp.minimum(jnp.maximum(x1, zero), wi - 1)
    y0c = jnp.minimum(jnp.maximum(y0, zero), hi - 1)
    y1c = jnp.minimum(jnp.maximum(y1, zero), hi - 1)

    row = lax.broadcasted_iota(jnp.int32, (n, NCOL), 0)
    b = (row >= 1024).astype(jnp.int32)
    rowbase = ab_ref[...] + hw8_ref[...] * b

    r0 = rowbase + y0c * wi
    r1 = rowbase + y1c * wi
    idx_ref[0] = r0 + x0c
    idx_ref[1] = r0 + x1c
    idx_ref[2] = r1 + x0c
    idx_ref[3] = r1 + x1c

    wx0a = wx0 * vx0
    wx1a = wx1 * vx1
    wy0a = wy0 * vy0 * attn
    wy1a = wy1 * vy1 * attn
    w_ref[0] = wx0a * wy0a
    w_ref[1] = wx1a * wy0a
    w_ref[2] = wx0a * wy1a
    w_ref[3] = wx1a * wy1a


def _k3_body(x_ref, w_ref, b_ref, o_ref):
    o_ref[...] = jnp.dot(x_ref[...], w_ref[...],
                         preferred_element_type=jnp.float32) + b_ref[...]


def _make_sc_gather(n_rows, table_rows):
    rpt = n_rows // NW
    mesh = plsc.VectorSubcoreMesh(core_axis_name="c", subcore_axis_name="s")

    gdn = lax.GatherDimensionNumbers(
        offset_dims=(), collapsed_slice_dims=(0,), start_index_map=(0,))

    def _bcast_lane(vec, lane_idx):
        return lax.gather(vec, lane_idx, dimension_numbers=gdn,
                          slice_sizes=(1,),
                          mode=lax.GatherScatterMode.PROMISE_IN_BOUNDS)

    @functools.partial(
        pl.kernel, mesh=mesh,
        out_type=jax.ShapeDtypeStruct((n_rows, EMBED_DIM), jnp.float32),
        scratch_types=[
            pltpu.VMEM((4, rpt, NCOL), jnp.int32),
            pltpu.VMEM((4, rpt, NCOL), jnp.float32),
            pltpu.VMEM((NBUF, 4, NCOL, HEAD_DIM), jnp.bfloat16),
            pltpu.VMEM((rpt, EMBED_DIM), jnp.float32),
        ] + [pltpu.SemaphoreType.DMA] * NBUF,
        compiler_params=pltpu.CompilerParams(use_tc_tiling_on_sc=False,
                                             needs_layout_passes=False),
    )
    def sc_kernel(table, idx4, w4, out, idx_v, w_v, rows_v, out_v, *sems):
        wid = lax.axis_index("s") * NC + lax.axis_index("c")
        base = wid * rpt
        lane_consts = [jnp.full((16, 1), t, jnp.int32) for t in range(16)]
        for c in range(4):
            pltpu.sync_copy(idx4.at[c, pl.ds(base, rpt)], idx_v.at[c])
            pltpu.sync_copy(w4.at[c, pl.ds(base, rpt)], w_v.at[c])

        def issue(g, slot, sem):
            for c in range(4):
                pltpu.async_copy(table.at[idx_v.at[c, g]], rows_v.at[slot, c], sem)

        def drain(slot, sem):
            for c in range(4):
                pltpu.make_async_copy(table.at[idx_v.at[c, 0]],
                                      rows_v.at[slot, c], sem).wait()

        for s in range(NBUF):
            issue(s, s, sems[s])

        def body(g, carry):
            slot = lax.rem(g, NBUF)
            for s in range(NBUF):
                @pl.when(slot == s)
                def _():
                    drain(s, sems[s])
            acc = [jnp.zeros((16,), jnp.float32) for _ in range(2 * NUM_HEADS)]
            for jj in range(NCOL // 16):
                wvecs = [w_v[c, g, pl.ds(jj * 16, 16)] for c in range(4)]
                for t in range(16):
                    j = jj * 16 + t
                    h = (j % (NUM_HEADS * NUM_LEVELS)) // NUM_LEVELS
                    for c in range(4):
                        wb = _bcast_lane(wvecs[c], lane_consts[t])
                        row_bf = rows_v[slot, c, j, :]
                        ve, vo = plsc.unpack(row_bf,
                                             format=plsc.PackFormat.INTERLEAVED)
                        acc[2 * h] = acc[2 * h] + wb * ve
                        acc[2 * h + 1] = acc[2 * h + 1] + wb * vo
            for h in range(NUM_HEADS):
                out_v[g, pl.ds(h * HEAD_DIM, 16)] = acc[2 * h]
                out_v[g, pl.ds(h * HEAD_DIM + 16, 16)] = acc[2 * h + 1]

            @pl.when(g + NBUF < rpt)
            def _():
                for s in range(NBUF):
                    @pl.when(slot == s)
                    def _():
                        issue(g + NBUF, s, sems[s])
            return carry

        lax.fori_loop(0, rpt, body, 0)
        pltpu.sync_copy(out_v, out.at[pl.ds(base, rpt)])

    return sc_kernel


def kernel(query, reference_points, value_feat_0, value_feat_1, value_feat_2,
           spatial_shapes, W_off, b_off, W_attn, b_attn, W_out, b_out):
    del spatial_shapes
    B, Q, D = query.shape
    n = B * Q

    q2d = query.reshape(n, D)
    refs = reference_points.reshape(n, 2)

    Wo = W_off.reshape(NUM_HEADS, NUM_LEVELS, NUM_POINTS, 2, D)
    Wo = Wo.transpose(3, 2, 0, 1, 4).reshape(2 * NCOL, D)
    bo = b_off.reshape(NUM_HEADS, NUM_LEVELS, NUM_POINTS, 2)
    bo = bo.transpose(3, 2, 0, 1).reshape(1, 2 * NCOL)
    Wa = W_attn.reshape(NUM_HEADS, NUM_LEVELS, NUM_POINTS, D)
    Wa = Wa.transpose(2, 0, 1, 3).reshape(NCOL, D)
    ba = b_attn.reshape(NUM_HEADS, NUM_LEVELS, NUM_POINTS)
    ba = ba.transpose(2, 0, 1).reshape(1, NCOL)

    tabs = []
    for vf in (value_feat_0, value_feat_1, value_feat_2):
        b_, c_, h_, w_ = vf.shape
        tabs.append(vf.reshape(B, NUM_HEADS, HEAD_DIM, h_, w_)
                    .transpose(0, 1, 3, 4, 2).reshape(-1, HEAD_DIM))
    table = jnp.concatenate(tabs, axis=0).astype(jnp.bfloat16)

    consts = tuple(jnp.asarray(a) for a in _col_constants(B))

    idx4, w4 = pl.pallas_call(
        _k1_body,
        out_shape=[
            jax.ShapeDtypeStruct((4, n, NCOL), jnp.int32),
            jax.ShapeDtypeStruct((4, n, NCOL), jnp.float32),
        ],
    )(q2d, refs, Wo.T, bo, Wa.T, ba, *consts)

    sampled = _make_sc_gather(n, table.shape[0])(table, idx4, w4)

    t_in_head = np.arange(HEAD_DIM)
    orig = np.where(t_in_head < 16, 2 * t_in_head, 2 * (t_in_head - 16) + 1)
    perm = (np.arange(NUM_HEADS)[:, None] * HEAD_DIM + orig[None, :]).reshape(-1)

    out = pl.pallas_call(
        _k3_body,
        out_shape=jax.ShapeDtypeStruct((n, D), jnp.float32),
    )(sampled, W_out.T[perm], b_out.reshape(1, D))

    return out.reshape(B, Q, D)


# speedup vs baseline: 1.9853x; 1.9853x over previous
import functools

import jax
import jax.numpy as jnp
import numpy as np
from jax import lax
from jax.experimental import pallas as pl
from jax.experimental.pallas import tpu as pltpu
from jax.experimental.pallas import tpu_sc as plsc

EMBED_DIM = 256
NUM_HEADS = 8
HEAD_DIM = EMBED_DIM // NUM_HEADS
NUM_LEVELS = 3
NUM_POINTS = 4
LEVEL_SHAPES = ((64, 64), (32, 32), (16, 16))

NCOL = NUM_POINTS * NUM_HEADS * NUM_LEVELS

NC, NS = 2, 16
NW = NC * NS
NBUF = 4


def _col_constants(batch):
    wf = np.zeros((1, NCOL), np.float32)
    hf = np.zeros((1, NCOL), np.float32)
    wi = np.zeros((1, NCOL), np.int32)
    hi = np.zeros((1, NCOL), np.int32)
    ab = np.zeros((1, NCOL), np.int32)
    hw8 = np.zeros((1, NCOL), np.int32)
    base = 0
    level_base = []
    for (H, W) in LEVEL_SHAPES:
        level_base.append(base)
        base += batch * NUM_HEADS * H * W
    for col in range(NCOL):
        l = col % NUM_LEVELS
        h = (col // NUM_LEVELS) % NUM_HEADS
        H, W = LEVEL_SHAPES[l]
        wf[0, col] = W
        hf[0, col] = H
        wi[0, col] = W
        hi[0, col] = H
        ab[0, col] = level_base[l] + h
        hw8[0, col] = NUM_HEADS * H * W
    return wf, hf, wi, hi, ab, hw8


def _k1_body(q_ref, r_ref, wo_ref, bo_ref, wa_ref, ba_ref, wf_ref, hf_ref,
             wi_ref, hi_ref, ab_ref, hw8_ref, vf0_ref, vf1_ref, vf2_ref,
             idx_ref, w_ref, tab_ref):
    n = q_ref.shape[0]
    q = q_ref[...]

    rr = lax.broadcasted_iota(jnp.int32, (EMBED_DIM, EMBED_DIM), 0)
    cc = lax.broadcasted_iota(jnp.int32, (EMBED_DIM, EMBED_DIM), 1)
    eye = (rr == cc).astype(jnp.bfloat16)
    off = 0
    for vf_ref in (vf0_ref, vf1_ref, vf2_ref):
        bsz, _, hw = vf_ref.shape
        for b in range(bsz):
            xs = vf_ref[b].astype(jnp.bfloat16)
            out = lax.dot_general(xs, eye, (((0,), (0,)), ((), ())),
                                  preferred_element_type=jnp.float32)
            tab_ref[pl.ds(off, hw)] = out.astype(jnp.bfloat16)
            off += hw
    off = jnp.dot(q, wo_ref[...], preferred_element_type=jnp.float32) + bo_ref[...]
    att = jnp.dot(q, wa_ref[...], preferred_element_type=jnp.float32) + ba_ref[...]

    g = NUM_HEADS * NUM_LEVELS
    a0, a1, a2, a3 = att[:, 0:g], att[:, g:2 * g], att[:, 2 * g:3 * g], att[:, 3 * g:4 * g]
    m = jnp.maximum(jnp.maximum(a0, a1), jnp.maximum(a2, a3))
    e0 = jnp.exp(a0 - m)
    e1 = jnp.exp(a1 - m)
    e2 = jnp.exp(a2 - m)
    e3 = jnp.exp(a3 - m)
    s = e0 + e1 + e2 + e3
    attn = jnp.concatenate([e0 / s, e1 / s, e2 / s, e3 / s], axis=1)

    refx = r_ref[:, 0:1]
    refy = r_ref[:, 1:2]
    locx = refx + jnp.tanh(off[:, 0:NCOL]) * 0.5
    locy = refy + jnp.tanh(off[:, NCOL:2 * NCOL]) * 0.5

    wf = wf_ref[...]
    hf = hf_ref[...]
    wi = wi_ref[...]
    hi = hi_ref[...]

    gx = locx * 2.0 - 1.0
    gy = locy * 2.0 - 1.0
    ix = ((gx + 1.0) * wf - 1.0) * 0.5
    iy = ((gy + 1.0) * hf - 1.0) * 0.5
    x0f = jnp.floor(ix)
    y0f = jnp.floor(iy)
    wx1 = ix - x0f
    wx0 = 1.0 - wx1
    wy1 = iy - y0f
    wy0 = 1.0 - wy1
    x0 = x0f.astype(jnp.int32)
    y0 = y0f.astype(jnp.int32)
    x1 = x0 + 1
    y1 = y0 + 1

    zero = jnp.zeros_like(x0)
    vx0 = ((x0 >= 0) & (x0 < wi)).astype(jnp.float32)
    vx1 = ((x1 >= 0) & (x1 < wi)).astype(jnp.float32)
    vy0 = ((y0 >= 0) & (y0 < hi)).astype(jnp.float32)
    vy1 = ((y1 >= 0) & (y1 < hi)).astype(jnp.float32)
    x0c = jnp.minimum(jnp.maximum(x0, zero), wi - 1)
    x1c = jnp.minimum(jnp.maximum(x1, zero), wi - 1)
    y0c = jnp.minimum(jnp.maximum(y0, zero), hi - 1)
    y1c = jnp.minimum(jnp.maximum(y1, zero), hi - 1)

    row = lax.broadcasted_iota(jnp.int32, (n, NCOL), 0)
    b = (row >= 1024).astype(jnp.int32)
    rowbase = ab_ref[...] + hw8_ref[...] * b

    wi8 = wi * 8
    r0 = rowbase + y0c * wi8
    r1 = rowbase + y1c * wi8
    idx_ref[0] = r0 + x0c * 8
    idx_ref[1] = r0 + x1c * 8
    idx_ref[2] = r1 + x0c * 8
    idx_ref[3] = r1 + x1c * 8

    wx0a = wx0 * vx0
    wx1a = wx1 * vx1
    wy0a = wy0 * vy0 * attn
    wy1a = wy1 * vy1 * attn
    w_ref[0] = wx0a * wy0a
    w_ref[1] = wx1a * wy0a
    w_ref[2] = wx0a * wy1a
    w_ref[3] = wx1a * wy1a


def _k3_body(x_ref, w_ref, b_ref, o_ref):
    o_ref[...] = jnp.dot(x_ref[...], w_ref[...],
                         preferred_element_type=jnp.float32) + b_ref[...]


def _make_sc_gather(n_rows, table_rows):
    rpt = n_rows // NW
    mesh = plsc.VectorSubcoreMesh(core_axis_name="c", subcore_axis_name="s")

    gdn = lax.GatherDimensionNumbers(
        offset_dims=(), collapsed_slice_dims=(0,), start_index_map=(0,))

    def _bcast_lane(vec, lane_idx):
        return lax.gather(vec, lane_idx, dimension_numbers=gdn,
                          slice_sizes=(1,),
                          mode=lax.GatherScatterMode.PROMISE_IN_BOUNDS)

    @functools.partial(
        pl.kernel, mesh=mesh,
        out_type=jax.ShapeDtypeStruct((n_rows, EMBED_DIM), jnp.float32),
        scratch_types=[
            pltpu.VMEM((4, rpt, NCOL), jnp.int32),
            pltpu.VMEM((4, rpt, NCOL), jnp.float32),
            pltpu.VMEM((NBUF, 4, NCOL, HEAD_DIM), jnp.bfloat16),
            pltpu.VMEM((rpt, EMBED_DIM), jnp.float32),
        ] + [pltpu.SemaphoreType.DMA] * NBUF,
        compiler_params=pltpu.CompilerParams(use_tc_tiling_on_sc=False,
                                             needs_layout_passes=False),
    )
    def sc_kernel(table, idx4, w4, out, idx_v, w_v, rows_v, out_v, *sems):
        wid = lax.axis_index("s") * NC + lax.axis_index("c")
        base = wid * rpt
        lane_consts = [jnp.full((16, 1), t, jnp.int32) for t in range(16)]
        for c in range(4):
            pltpu.sync_copy(idx4.at[c, pl.ds(base, rpt)], idx_v.at[c])
            pltpu.sync_copy(w4.at[c, pl.ds(base, rpt)], w_v.at[c])

        def issue(g, slot, sem):
            for c in range(4):
                pltpu.async_copy(table.at[idx_v.at[c, g]], rows_v.at[slot, c], sem)

        def drain(slot, sem):
            for c in range(4):
                pltpu.make_async_copy(table.at[idx_v.at[c, 0]],
                                      rows_v.at[slot, c], sem).wait()

        for s in range(NBUF):
            issue(s, s, sems[s])

        def body(g, carry):
            slot = lax.rem(g, NBUF)
            for s in range(NBUF):
                @pl.when(slot == s)
                def _():
                    drain(s, sems[s])
            acc = [jnp.zeros((16,), jnp.float32) for _ in range(2 * NUM_HEADS)]
            for jj in range(NCOL // 16):
                wvecs = [w_v[c, g, pl.ds(jj * 16, 16)] for c in range(4)]
                for t in range(16):
                    j = jj * 16 + t
                    h = (j % (NUM_HEADS * NUM_LEVELS)) // NUM_LEVELS
                    for c in range(4):
                        wb = _bcast_lane(wvecs[c], lane_consts[t])
                        row_bf = rows_v[slot, c, j, :]
                        ve, vo = plsc.unpack(row_bf,
                                             format=plsc.PackFormat.INTERLEAVED)
                        acc[2 * h] = acc[2 * h] + wb * ve
                        acc[2 * h + 1] = acc[2 * h + 1] + wb * vo
            for h in range(NUM_HEADS):
                out_v[g, pl.ds(h * HEAD_DIM, 16)] = acc[2 * h]
                out_v[g, pl.ds(h * HEAD_DIM + 16, 16)] = acc[2 * h + 1]

            @pl.when(g + NBUF < rpt)
            def _():
                for s in range(NBUF):
                    @pl.when(slot == s)
                    def _():
                        issue(g + NBUF, s, sems[s])
            return carry

        lax.fori_loop(0, rpt, body, 0)
        pltpu.sync_copy(out_v, out.at[pl.ds(base, rpt)])

    return sc_kernel


def kernel(query, reference_points, value_feat_0, value_feat_1, value_feat_2,
           spatial_shapes, W_off, b_off, W_attn, b_attn, W_out, b_out):
    del spatial_shapes
    B, Q, D = query.shape
    n = B * Q

    q2d = query.reshape(n, D)
    refs = reference_points.reshape(n, 2)

    Wo = W_off.reshape(NUM_HEADS, NUM_LEVELS, NUM_POINTS, 2, D)
    Wo = Wo.transpose(3, 2, 0, 1, 4).reshape(2 * NCOL, D)
    bo = b_off.reshape(NUM_HEADS, NUM_LEVELS, NUM_POINTS, 2)
    bo = bo.transpose(3, 2, 0, 1).reshape(1, 2 * NCOL)
    Wa = W_attn.reshape(NUM_HEADS, NUM_LEVELS, NUM_POINTS, D)
    Wa = Wa.transpose(2, 0, 1, 3).reshape(NCOL, D)
    ba = b_attn.reshape(NUM_HEADS, NUM_LEVELS, NUM_POINTS)
    ba = ba.transpose(2, 0, 1).reshape(1, NCOL)

    consts = tuple(jnp.asarray(a) for a in _col_constants(B))

    vfs = [vf.reshape(B, EMBED_DIM, -1)
           for vf in (value_feat_0, value_feat_1, value_feat_2)]
    n_pix = sum(v.shape[2] for v in vfs) * B
    n_rows_table = n_pix * NUM_HEADS

    idx4, w4, table256 = pl.pallas_call(
        _k1_body,
        out_shape=[
            jax.ShapeDtypeStruct((4, n, NCOL), jnp.int32),
            jax.ShapeDtypeStruct((4, n, NCOL), jnp.float32),
            jax.ShapeDtypeStruct((n_pix, EMBED_DIM), jnp.bfloat16),
        ],
    )(q2d, refs, Wo.T, bo, Wa.T, ba, *consts, *vfs)

    table = table256.reshape(n_rows_table, HEAD_DIM)
    sampled = _make_sc_gather(n, n_rows_table)(table, idx4, w4)

    t_in_head = np.arange(HEAD_DIM)
    orig = np.where(t_in_head < 16, 2 * t_in_head, 2 * (t_in_head - 16) + 1)
    perm = (np.arange(NUM_HEADS)[:, None] * HEAD_DIM + orig[None, :]).reshape(-1)

    out = pl.pallas_call(
        _k3_body,
        out_shape=jax.ShapeDtypeStruct((n, D), jnp.float32),
    )(sampled, W_out.T[perm], b_out.reshape(1, D))

    return out.reshape(B, Q, D)
